# baseline (speedup 1.0000x reference)
import jax
import jax.numpy as jnp
from jax.experimental import pallas as pl
from jax.experimental.pallas import tpu as pltpu

_NB = 2
_N = 10000
_FT = 128
_HID = 64
_TILE = 200


def _project_kernel(f_ref, s_ref, w_ref, z_ref):
    w = w_ref[0]
    z_ref[0, :, 0:_HID] = jnp.dot(f_ref[0], w, preferred_element_type=jnp.float32)
    z_ref[0, :, _HID:2 * _HID] = jnp.dot(s_ref[0], w, preferred_element_type=jnp.float32)


def _scores(h1, h2, wb_ref):
    cm = jax.nn.sigmoid(jnp.mean(h1, axis=0, keepdims=True))
    v = jax.lax.dot_general(cm, wb_ref[...], (((1,), (1,)), ((), ())),
                            preferred_element_type=jnp.float32)
    sc1 = jax.lax.dot_general(h1, v, (((1,), (1,)), ((), ())),
                              preferred_element_type=jnp.float32)
    sc2 = jax.lax.dot_general(h2, v, (((1,), (1,)), ((), ())),
                              preferred_element_type=jnp.float32)
    return jnp.concatenate([sc1, sc2], axis=1)


def _fused_kernel(z_ref, a_ref, wb_ref, hm_ref, sc_ref, reg_ref, h_scr):
    g = pl.program_id(0)
    t = pl.program_id(1)
    nt = pl.num_programs(1)

    y = jnp.dot(a_ref[0], z_ref[0], preferred_element_type=jnp.float32)
    h_scr[pl.ds(g * _N + t * _TILE, _TILE), :] = jnp.maximum(y, 0.0)

    @pl.when(jnp.logical_and(g == 0, t == nt - 1))
    def _():
        sc_ref[:, 0:2] = _scores(h_scr[0:_N, 0:_HID],
                                 h_scr[0:_N, _HID:2 * _HID], wb_ref)

    @pl.when(jnp.logical_and(g == 1, t == nt - 1))
    def _():
        sc_ref[:, 2:4] = _scores(h_scr[_N:2 * _N, 0:_HID],
                                 h_scr[_N:2 * _N, _HID:2 * _HID], wb_ref)
        hm = hm_ref[0]
        h1a = 0.5 * (h_scr[0:_N, 0:_HID] + h_scr[_N:2 * _N, 0:_HID])
        h2a = 0.5 * (h_scr[0:_N, _HID:2 * _HID] + h_scr[_N:2 * _N, _HID:2 * _HID])
        pos = jnp.sum((hm - h1a) ** 2)
        neg = jnp.sum((hm - h2a) ** 2)
        reg_ref[:, :] = jnp.reshape(pos - neg, (1, 1))


def kernel(feature, adj, shuf, sparse, msk, samp_bias1, samp_bias2, W_gcn, W_bil, b_bil, H):
    f = feature.reshape(_NB, _N, _FT)
    s = shuf.reshape(_NB, _N, _FT)
    a = adj.reshape(_NB, _N, _N)

    z = pl.pallas_call(
        _project_kernel,
        grid=(_NB,),
        in_specs=[
            pl.BlockSpec((1, _N, _FT), lambda g: (g, 0, 0)),
            pl.BlockSpec((1, _N, _FT), lambda g: (g, 0, 0)),
            pl.BlockSpec((1, _FT, _HID), lambda g: (g, 0, 0)),
        ],
        out_specs=pl.BlockSpec((1, _N, 2 * _HID), lambda g: (g, 0, 0)),
        out_shape=jax.ShapeDtypeStruct((_NB, _N, 2 * _HID), jnp.float32),
    )(f, s, W_gcn)

    sc, reg = pl.pallas_call(
        _fused_kernel,
        grid=(_NB, _N // _TILE),
        in_specs=[
            pl.BlockSpec((1, _N, 2 * _HID), lambda g, t: (g, 0, 0)),
            pl.BlockSpec((1, _TILE, _N), lambda g, t: (g, t, 0)),
            pl.BlockSpec((_HID, _HID), lambda g, t: (0, 0)),
            pl.BlockSpec((1, _N, _HID), lambda g, t: (0, 0, 0)),
        ],
        out_specs=[
            pl.BlockSpec((_N, 4), lambda g, t: (0, 0)),
            pl.BlockSpec((1, 1), lambda g, t: (0, 0)),
        ],
        out_shape=[
            jax.ShapeDtypeStruct((_N, 4), jnp.float32),
            jax.ShapeDtypeStruct((1, 1), jnp.float32),
        ],
        scratch_shapes=[
            pltpu.VMEM((_NB * _N, 2 * _HID), jnp.float32),
        ],
    )(z, a, W_bil, H)

    logits = jnp.transpose(sc).reshape(_NB, 1, 2 * _N)
    logits = logits + jnp.concatenate([samp_bias1, samp_bias2], axis=1)[None] + b_bil
    reg_loss = reg[0, 0]
    return (logits, reg_loss)


# speedup vs baseline: 1.0177x; 1.0177x over previous
import jax
import jax.numpy as jnp
from jax.experimental import pallas as pl
from jax.experimental.pallas import tpu as pltpu

_NB = 2
_N = 10000
_FT = 128
_HID = 64
_TILE = 200
_STEP = 2 * _TILE


def _propagate_kernel(f_ref, s_ref, w_ref, a0_ref, a1_ref, h_ref, z_scr):
    t = pl.program_id(1)

    @pl.when(t == 0)
    def _():
        w = w_ref[0]
        z_scr[:, 0:_HID] = jnp.dot(f_ref[0], w, preferred_element_type=jnp.float32)
        z_scr[:, _HID:2 * _HID] = jnp.dot(s_ref[0], w, preferred_element_type=jnp.float32)

    z = z_scr[...]
    y0 = jnp.dot(a0_ref[0], z, preferred_element_type=jnp.float32)
    y1 = jnp.dot(a1_ref[0], z, preferred_element_type=jnp.float32)
    h_ref[0, 0:_TILE, :] = jnp.maximum(y0, 0.0)
    h_ref[0, _TILE:_STEP, :] = jnp.maximum(y1, 0.0)


def _epilogue_kernel(h_ref, wb_ref, hmat_ref, sc_ref, reg_ref):
    wb = wb_ref[...]
    hm = hmat_ref[0]
    for g in range(_NB):
        h1 = h_ref[g, :, 0:_HID]
        h2 = h_ref[g, :, _HID:2 * _HID]
        cm = jax.nn.sigmoid(jnp.mean(h1, axis=0, keepdims=True))
        v = jax.lax.dot_general(cm, wb, (((1,), (1,)), ((), ())),
                                preferred_element_type=jnp.float32)
        sc1 = jax.lax.dot_general(h1, v, (((1,), (1,)), ((), ())),
                                  preferred_element_type=jnp.float32)
        sc2 = jax.lax.dot_general(h2, v, (((1,), (1,)), ((), ())),
                                  preferred_element_type=jnp.float32)
        sc_ref[:, 2 * g:2 * g + 1] = sc1
        sc_ref[:, 2 * g + 1:2 * g + 2] = sc2
    h1a = 0.5 * (h_ref[0, :, 0:_HID] + h_ref[1, :, 0:_HID])
    h2a = 0.5 * (h_ref[0, :, _HID:2 * _HID] + h_ref[1, :, _HID:2 * _HID])
    pos = jnp.sum((hm - h1a) ** 2)
    neg = jnp.sum((hm - h2a) ** 2)
    reg_ref[:, :] = jnp.reshape(pos - neg, (1, 1))


def kernel(feature, adj, shuf, sparse, msk, samp_bias1, samp_bias2, W_gcn, W_bil, b_bil, H):
    f = feature.reshape(_NB, _N, _FT)
    s = shuf.reshape(_NB, _N, _FT)
    a = adj.reshape(_NB, _N, _N)

    h = pl.pallas_call(
        _propagate_kernel,
        grid=(_NB, _N // _STEP),
        in_specs=[
            pl.BlockSpec((1, _N, _FT), lambda g, t: (g, 0, 0)),
            pl.BlockSpec((1, _N, _FT), lambda g, t: (g, 0, 0)),
            pl.BlockSpec((1, _FT, _HID), lambda g, t: (g, 0, 0)),
            pl.BlockSpec((1, _TILE, _N), lambda g, t: (g, 2 * t, 0)),
            pl.BlockSpec((1, _TILE, _N), lambda g, t: (g, 2 * t + 1, 0)),
        ],
        out_specs=pl.BlockSpec((1, _STEP, 2 * _HID), lambda g, t: (g, t, 0)),
        out_shape=jax.ShapeDtypeStruct((_NB, _N, 2 * _HID), jnp.float32),
        scratch_shapes=[pltpu.VMEM((_N, 2 * _HID), jnp.float32)],
    )(f, s, W_gcn, a, a)

    sc, reg = pl.pallas_call(
        _epilogue_kernel,
        in_specs=[
            pl.BlockSpec((_NB, _N, 2 * _HID), lambda: (0, 0, 0)),
            pl.BlockSpec((_HID, _HID), lambda: (0, 0)),
            pl.BlockSpec((1, _N, _HID), lambda: (0, 0, 0)),
        ],
        out_specs=[
            pl.BlockSpec((_N, 4), lambda: (0, 0)),
            pl.BlockSpec((1, 1), lambda: (0, 0)),
        ],
        out_shape=[
            jax.ShapeDtypeStruct((_N, 4), jnp.float32),
            jax.ShapeDtypeStruct((1, 1), jnp.float32),
        ],
    )(h, W_bil, H)

    logits = jnp.transpose(sc).reshape(_NB, 1, 2 * _N)
    logits = logits + jnp.concatenate([samp_bias1, samp_bias2], axis=1)[None] + b_bil
    reg_loss = reg[0, 0]
    return (logits, reg_loss)


# speedup vs baseline: 1.0314x; 1.0135x over previous
import jax
import jax.numpy as jnp
from jax.experimental import pallas as pl
from jax.experimental.pallas import tpu as pltpu

_NB = 2
_N = 10000
_FT = 128
_HID = 64
_TILE = 80
_NSTREAM = 5
_STEP = _NSTREAM * _TILE


def _propagate_kernel(f_ref, s_ref, w_ref, *refs):
    a_refs = refs[:_NSTREAM]
    h_ref = refs[_NSTREAM]
    z_scr = refs[_NSTREAM + 1]
    t = pl.program_id(1)

    @pl.when(t == 0)
    def _():
        w = w_ref[0]
        z_scr[:, 0:_HID] = jnp.dot(f_ref[0], w, preferred_element_type=jnp.float32)
        z_scr[:, _HID:2 * _HID] = jnp.dot(s_ref[0], w, preferred_element_type=jnp.float32)

    z = z_scr[...]
    for i in range(_NSTREAM):
        y = jnp.dot(a_refs[i][0], z, preferred_element_type=jnp.float32)
        h_ref[0, i * _TILE:(i + 1) * _TILE, :] = jnp.maximum(y, 0.0)


def _epilogue_kernel(h_ref, wb_ref, hmat_ref, sc_ref, reg_ref):
    wb = wb_ref[...]
    hm = hmat_ref[0]
    for g in range(_NB):
        h1 = h_ref[g, :, 0:_HID]
        h2 = h_ref[g, :, _HID:2 * _HID]
        cm = jax.nn.sigmoid(jnp.mean(h1, axis=0, keepdims=True))
        v = jax.lax.dot_general(cm, wb, (((1,), (1,)), ((), ())),
                                preferred_element_type=jnp.float32)
        sc1 = jax.lax.dot_general(h1, v, (((1,), (1,)), ((), ())),
                                  preferred_element_type=jnp.float32)
        sc2 = jax.lax.dot_general(h2, v, (((1,), (1,)), ((), ())),
                                  preferred_element_type=jnp.float32)
        sc_ref[:, 2 * g:2 * g + 1] = sc1
        sc_ref[:, 2 * g + 1:2 * g + 2] = sc2
    h1a = 0.5 * (h_ref[0, :, 0:_HID] + h_ref[1, :, 0:_HID])
    h2a = 0.5 * (h_ref[0, :, _HID:2 * _HID] + h_ref[1, :, _HID:2 * _HID])
    pos = jnp.sum((hm - h1a) ** 2)
    neg = jnp.sum((hm - h2a) ** 2)
    reg_ref[:, :] = jnp.reshape(pos - neg, (1, 1))


def kernel(feature, adj, shuf, sparse, msk, samp_bias1, samp_bias2, W_gcn, W_bil, b_bil, H):
    f = feature.reshape(_NB, _N, _FT)
    s = shuf.reshape(_NB, _N, _FT)
    a = adj.reshape(_NB, _N, _N)

    h = pl.pallas_call(
        _propagate_kernel,
        grid=(_NB, _N // _STEP),
        in_specs=[
            pl.BlockSpec((1, _N, _FT), lambda g, t: (g, 0, 0)),
            pl.BlockSpec((1, _N, _FT), lambda g, t: (g, 0, 0)),
            pl.BlockSpec((1, _FT, _HID), lambda g, t: (g, 0, 0)),
        ] + [
            pl.BlockSpec((1, _TILE, _N),
                         (lambda i: lambda g, t: (g, _NSTREAM * t + i, 0))(i))
            for i in range(_NSTREAM)
        ],
        out_specs=pl.BlockSpec((1, _STEP, 2 * _HID), lambda g, t: (g, t, 0)),
        out_shape=jax.ShapeDtypeStruct((_NB, _N, 2 * _HID), jnp.float32),
        scratch_shapes=[pltpu.VMEM((_N, 2 * _HID), jnp.float32)],
    )(f, s, W_gcn, *([a] * _NSTREAM))

    sc, reg = pl.pallas_call(
        _epilogue_kernel,
        in_specs=[
            pl.BlockSpec((_NB, _N, 2 * _HID), lambda: (0, 0, 0)),
            pl.BlockSpec((_HID, _HID), lambda: (0, 0)),
            pl.BlockSpec((1, _N, _HID), lambda: (0, 0, 0)),
        ],
        out_specs=[
            pl.BlockSpec((_N, 4), lambda: (0, 0)),
            pl.BlockSpec((1, 1), lambda: (0, 0)),
        ],
        out_shape=[
            jax.ShapeDtypeStruct((_N, 4), jnp.float32),
            jax.ShapeDtypeStruct((1, 1), jnp.float32),
        ],
    )(h, W_bil, H)

    logits = jnp.transpose(sc).reshape(_NB, 1, 2 * _N)
    logits = logits + jnp.concatenate([samp_bias1, samp_bias2], axis=1)[None] + b_bil
    reg_loss = reg[0, 0]
    return (logits, reg_loss)
